# Initial kernel scaffold; baseline (speedup 1.0000x reference)
import jax
import jax.numpy as jnp
from jax.experimental import pallas as pl


def kernel(x, edge_index, edge_attr, W_edge, b_edge):
    raise NotImplementedError("write your pallas kernel here")


# speedup vs baseline: 3.5008x; 3.5008x over previous
import functools

import jax
import jax.numpy as jnp
from jax import lax
from jax.experimental import pallas as pl
from jax.experimental.pallas import tpu as pltpu
from jax.experimental.pallas import tpu_sc as plsc

N_NODES = 10000
N_EDGES = 320000
D = 128
R = 16

NC = 2
NS = 16
DH = D // NC
EPT = N_EDGES // NS
C = 80
NCHUNK = EPT // C
NP = 10240
RPT = NP // NS
ZR = 128


def _sc_scatter(xs, src, dst, edge_attr):
    mesh = plsc.VectorSubcoreMesh(
        core_axis_name="c", subcore_axis_name="s", num_cores=NC, num_subcores=NS
    )

    @functools.partial(
        pl.kernel,
        mesh=mesh,
        compiler_params=pltpu.CompilerParams(use_tc_tiling_on_sc=False),
        out_type=[
            jax.ShapeDtypeStruct((NC, NP, DH), jnp.float32),
            jax.ShapeDtypeStruct((NC, NP, R), jnp.float32),
        ],
        scratch_types=[
            pltpu.VMEM((NCHUNK, C), jnp.int32),
            pltpu.VMEM((NCHUNK, C), jnp.int32),
            pltpu.VMEM((C, DH), jnp.float32),
            pltpu.VMEM((C, R), jnp.float32),
            pltpu.VMEM((ZR, DH), jnp.float32),
            pltpu.VMEM((ZR, R), jnp.float32),
            pltpu.VMEM_SHARED((NP, DH), jnp.float32),
            pltpu.VMEM_SHARED((NP, R), jnp.float32),
            pltpu.SemaphoreType.DMA,
        ],
    )
    def k(x_hbm, src_hbm, dst_hbm, attr_hbm, out_hbm, outa_hbm,
          src_v, dst_v, rows_v, attr_v, z_v, za_v, acc, acca, sem):
        cid = lax.axis_index("c")
        sid = lax.axis_index("s")

        def zrow(i, _):
            def zcol(j, _):
                z_v[i, pl.ds(j * 16, 16)] = jnp.zeros((16,), jnp.float32)
                return 0
            lax.fori_loop(0, DH // 16, zcol, 0)
            za_v[i, pl.ds(0, 16)] = jnp.zeros((16,), jnp.float32)
            return 0
        lax.fori_loop(0, ZR, zrow, 0)

        row0 = sid * RPT
        def zacc(i, _):
            pltpu.sync_copy(z_v, acc.at[pl.ds(row0 + i * ZR, ZR)])
            pltpu.sync_copy(za_v, acca.at[pl.ds(row0 + i * ZR, ZR)])
            return 0
        lax.fori_loop(0, RPT // ZR, zacc, 0)

        pltpu.sync_copy(src_hbm.at[sid], src_v)
        pltpu.sync_copy(dst_hbm.at[sid], dst_v)

        plsc.subcore_barrier()

        ebase = sid * EPT

        def chunk(j, _):
            pltpu.async_copy(x_hbm.at[cid].at[src_v.at[j]], rows_v, sem).wait()
            pltpu.sync_copy(rows_v, acc.at[dst_v.at[j]], add=True)
            return 0
        lax.fori_loop(0, NCHUNK, chunk, 0)

        def achunk(jj, _):
            j = 2 * jj + cid
            pltpu.sync_copy(attr_hbm.at[pl.ds(ebase + j * C, C)], attr_v)
            pltpu.sync_copy(attr_v, acca.at[dst_v.at[j]], add=True)
            return 0
        lax.fori_loop(0, NCHUNK // 2, achunk, 0)

        plsc.subcore_barrier()

        pltpu.sync_copy(acc.at[pl.ds(row0, RPT)],
                        out_hbm.at[cid, pl.ds(row0, RPT)])
        pltpu.sync_copy(acca.at[pl.ds(row0, RPT)],
                        outa_hbm.at[cid, pl.ds(row0, RPT)])

    return k(xs, src, dst, edge_attr)


BN = 1024


def _combine(p, a, w):
    def body(p_ref, a_ref, w_ref, o_ref):
        asum = a_ref[0] + a_ref[1]
        aw = jnp.dot(asum, w_ref[...], preferred_element_type=jnp.float32)
        px = jnp.concatenate([p_ref[0], p_ref[1]], axis=-1)
        o_ref[...] = px + aw

    return pl.pallas_call(
        body,
        grid=(NP // BN,),
        in_specs=[
            pl.BlockSpec((NC, BN, DH), lambda i: (0, i, 0)),
            pl.BlockSpec((NC, BN, R), lambda i: (0, i, 0)),
            pl.BlockSpec((R, D), lambda i: (0, 0)),
        ],
        out_specs=pl.BlockSpec((BN, D), lambda i: (i, 0)),
        out_shape=jax.ShapeDtypeStruct((NP, D), jnp.float32),
    )(p, a, w)


def kernel(x, edge_index, edge_attr, W_edge, b_edge):
    x_b = x + b_edge[None, :]
    xs = jnp.transpose(x_b.reshape(N_NODES, NC, DH), (1, 0, 2))
    ei = edge_index.astype(jnp.int32)
    src = ei[0].reshape(NS, NCHUNK, C)
    dst = ei[1].reshape(NS, NCHUNK, C)
    p, a = _sc_scatter(xs, src, dst, edge_attr)
    return _combine(p, a, W_edge)[:N_NODES]
